# baseline (speedup 1.0000x reference)
import functools

import jax
import jax.numpy as jnp
from jax import lax
from jax.experimental import pallas as pl
from jax.experimental.pallas import tpu as pltpu
from jax.experimental.pallas import tpu_sc as plsc

N = 10000
D = 128
E = 320000
NC, NS = 2, 16
NW = NC * NS
NPAD = 10240
RPT = NPAD // NS
C = 128
K = 8

CPW = 80
NCH_E = NW * CPW
SR_E = CPW // K

CPW_D = 160
NCH_D = NW * CPW_D
SR_D = CPW_D // K

TCB = 2048
TCG = NPAD // TCB


def _deg_body(edges_hbm, deg_out, ibuf, ones_v, stage_v, deg_acc, isem, ssem):
    c = lax.axis_index("c")
    s = lax.axis_index("s")
    wid = s * NC + c
    row0 = s * RPT
    ch0 = wid * CPW_D
    for j in range(C // 16):
        ones_v[pl.ds(j * 16, 16)] = jnp.ones((16,), jnp.float32)
    for j in range(RPT // 16):
        stage_v[pl.ds(j * 16, 16)] = jnp.zeros((16,), jnp.float32)
    pltpu.sync_copy(stage_v, deg_acc.at[pl.ds(row0, RPT)])
    plsc.subcore_barrier()

    pltpu.async_copy(edges_hbm.at[pl.ds(ch0, K)], ibuf.at[0], isem)

    def group(t, ib):
        nib = 1 - ib

        @pl.when(t < SR_D - 1)
        def _():
            pltpu.async_copy(
                edges_hbm.at[pl.ds(ch0 + (t + 1) * K, K)], ibuf.at[nib], isem)

        pltpu.make_async_copy(
            edges_hbm.at[pl.ds(ch0, K)], ibuf.at[ib], isem).wait()
        for j in range(K):
            pltpu.async_copy(ones_v, deg_acc.at[ibuf.at[ib, j]], ssem, add=True)
        for j in range(K):
            pltpu.make_async_copy(
                ones_v, deg_acc.at[ibuf.at[ib, j]], ssem).wait()

    def body(u, carry):
        group(2 * u, 0)
        group(2 * u + 1, 1)
        return carry

    lax.fori_loop(0, SR_D // 2, body, 0)
    plsc.subcore_barrier()
    pltpu.sync_copy(deg_acc.at[pl.ds(row0, RPT)], stage_v)
    pltpu.sync_copy(stage_v, deg_out.at[c, pl.ds(row0, RPT)])


def _agg_body(h_hbm, src_hbm, dst_hbm, out_hbm,
              sbuf, dbuf, rows, acc, isem, gsem, ssem):
    c = lax.axis_index("c")
    s = lax.axis_index("s")
    wid = s * NC + c
    row0 = s * RPT
    ch0 = wid * CPW

    for i in range(16):
        for j in range(D // 16):
            rows[0, i, pl.ds(j * 16, 16)] = jnp.zeros((16,), jnp.float32)

    def zbody(k, carry):
        pltpu.sync_copy(rows.at[0, pl.ds(0, 16)],
                        acc.at[pl.ds(row0 + k * 16, 16)])
        return carry

    lax.fori_loop(0, RPT // 16, zbody, 0)
    plsc.subcore_barrier()

    pltpu.async_copy(src_hbm.at[pl.ds(ch0, K)], sbuf.at[0], isem)
    pltpu.async_copy(dst_hbm.at[pl.ds(ch0, K)], dbuf.at[0], isem)

    def group(t, ib):
        nib = 1 - ib

        @pl.when(t < SR_E - 1)
        def _():
            base = ch0 + (t + 1) * K
            pltpu.async_copy(src_hbm.at[pl.ds(base, K)], sbuf.at[nib], isem)
            pltpu.async_copy(dst_hbm.at[pl.ds(base, K)], dbuf.at[nib], isem)

        pltpu.make_async_copy(
            src_hbm.at[pl.ds(ch0, K)], sbuf.at[ib], isem).wait()
        pltpu.make_async_copy(
            dst_hbm.at[pl.ds(ch0, K)], dbuf.at[ib], isem).wait()

        pltpu.async_copy(h_hbm.at[sbuf.at[ib, 0]], rows.at[0], gsem)
        for j in range(K):
            b = j % 2
            if j + 1 < K:
                if j >= 1:
                    pltpu.make_async_copy(
                        rows.at[(j + 1) % 2],
                        acc.at[dbuf.at[ib, j - 1]], ssem).wait()
                pltpu.async_copy(
                    h_hbm.at[sbuf.at[ib, j + 1]], rows.at[(j + 1) % 2], gsem)
            pltpu.make_async_copy(
                h_hbm.at[sbuf.at[ib, j]], rows.at[b], gsem).wait()
            pltpu.async_copy(rows.at[b], acc.at[dbuf.at[ib, j]], ssem, add=True)
        pltpu.make_async_copy(
            rows.at[0], acc.at[dbuf.at[ib, K - 2]], ssem).wait()
        pltpu.make_async_copy(
            rows.at[1], acc.at[dbuf.at[ib, K - 1]], ssem).wait()

    def body(u, carry):
        group(2 * u, 0)
        group(2 * u + 1, 1)
        return carry

    lax.fori_loop(0, SR_E // 2, body, 0)
    plsc.subcore_barrier()
    for k in range(RPT // C):
        pltpu.sync_copy(acc.at[pl.ds(row0 + k * C, C)], rows.at[0])
        pltpu.sync_copy(rows.at[0], out_hbm.at[c, pl.ds(row0 + k * C, C)])


def _sc_degree(edges2d):
    mesh = plsc.VectorSubcoreMesh(core_axis_name="c", subcore_axis_name="s")
    f = pl.kernel(
        _deg_body,
        out_type=jax.ShapeDtypeStruct((NC, NPAD), jnp.float32),
        mesh=mesh,
        scratch_types=[
            pltpu.VMEM((2, K, C), jnp.int32),
            pltpu.VMEM((C,), jnp.float32),
            pltpu.VMEM((RPT,), jnp.float32),
            pltpu.VMEM_SHARED((NPAD,), jnp.float32),
            pltpu.SemaphoreType.DMA,
            pltpu.SemaphoreType.DMA,
        ],
    )
    return f(edges2d)


def _sc_aggregate(h, src2d, dst2d):
    mesh = plsc.VectorSubcoreMesh(core_axis_name="c", subcore_axis_name="s")
    f = pl.kernel(
        _agg_body,
        out_type=jax.ShapeDtypeStruct((NC, NPAD, D), jnp.float32),
        mesh=mesh,
        scratch_types=[
            pltpu.VMEM((2, K, C), jnp.int32),
            pltpu.VMEM((2, K, C), jnp.int32),
            pltpu.VMEM((2, C, D), jnp.float32),
            pltpu.VMEM_SHARED((NPAD, D), jnp.float32),
            pltpu.SemaphoreType.DMA,
            pltpu.SemaphoreType.DMA,
            pltpu.SemaphoreType.DMA,
        ],
    )
    return f(h, src2d, dst2d)


def _prep_body(deg_ref, emb_ref, norm_ref, h_ref):
    d = deg_ref[0] + deg_ref[1]
    n = lax.rsqrt(jnp.maximum(d, 1.0))
    norm_ref[...] = n
    h_ref[...] = emb_ref[...] * n


def _tc_prep(deg2, emb):
    return pl.pallas_call(
        _prep_body,
        grid=(TCG,),
        in_specs=[
            pl.BlockSpec((NC, TCB, 1), lambda i: (0, i, 0)),
            pl.BlockSpec((TCB, D), lambda i: (i, 0)),
        ],
        out_specs=[
            pl.BlockSpec((TCB, 1), lambda i: (i, 0)),
            pl.BlockSpec((TCB, D), lambda i: (i, 0)),
        ],
        out_shape=[
            jax.ShapeDtypeStruct((NPAD, 1), jnp.float32),
            jax.ShapeDtypeStruct((NPAD, D), jnp.float32),
        ],
    )(deg2, emb)


def _comb_body(scale, p_ref, norm_ref, s_ref, sout_ref, hout_ref):
    nrm = norm_ref[...]
    e = (p_ref[0] + p_ref[1]) * nrm
    sout_ref[...] = (s_ref[...] + e) * scale
    hout_ref[...] = e * nrm


def _tc_combine(p, norm, s_in, scale):
    return pl.pallas_call(
        functools.partial(_comb_body, scale),
        grid=(TCG,),
        in_specs=[
            pl.BlockSpec((NC, TCB, D), lambda i: (0, i, 0)),
            pl.BlockSpec((TCB, 1), lambda i: (i, 0)),
            pl.BlockSpec((TCB, D), lambda i: (i, 0)),
        ],
        out_specs=[
            pl.BlockSpec((TCB, D), lambda i: (i, 0)),
            pl.BlockSpec((TCB, D), lambda i: (i, 0)),
        ],
        out_shape=[
            jax.ShapeDtypeStruct((NPAD, D), jnp.float32),
            jax.ShapeDtypeStruct((NPAD, D), jnp.float32),
        ],
    )(p, norm, s_in)


def kernel(entity_embedding, edge_index):
    src = edge_index[0]
    dst = edge_index[1]
    src2d = jnp.pad(src, (0, NCH_E * C - E)).reshape(NCH_E, C)
    dst2d = jnp.pad(dst, (0, NCH_E * C - E),
                    constant_values=NPAD - 1).reshape(NCH_E, C)
    edges2d = jnp.pad(edge_index.reshape(2 * E), (0, NCH_D * C - 2 * E),
                      constant_values=NPAD - 1).reshape(NCH_D, C)
    emb = jnp.pad(entity_embedding, ((0, NPAD - N), (0, 0)))

    deg2 = _sc_degree(edges2d).reshape(NC, NPAD, 1)
    norm, h = _tc_prep(deg2, emb)

    s_acc = emb
    for layer in range(3):
        p = _sc_aggregate(h, src2d, dst2d)
        scale = 0.25 if layer == 2 else 1.0
        s_acc, h = _tc_combine(p, norm, s_acc, scale)

    return s_acc[:N]


# speedup vs baseline: 3.4795x; 3.4795x over previous
import functools

import jax
import jax.numpy as jnp
from jax import lax
from jax.experimental import pallas as pl
from jax.experimental.pallas import tpu as pltpu
from jax.experimental.pallas import tpu_sc as plsc

N = 10000
D = 128
E = 320000
NC, NS = 2, 16
NW = NC * NS
NPAD = 10240
RPT = NPAD // NS
C = 128
K = 8

CPW = 80
NCH_E = NW * CPW
SR_E = CPW // K

CPW_D = 160
NCH_D = NW * CPW_D
SR_D = CPW_D // K

TCB = 2048
TCG = NPAD // TCB


def _deg_body(edges_hbm, deg_out, ibuf, ones_v, stage_v, deg_acc, isem, ssem):
    c = lax.axis_index("c")
    s = lax.axis_index("s")
    wid = s * NC + c
    row0 = s * RPT
    ch0 = wid * CPW_D
    for j in range(C // 16):
        ones_v[pl.ds(j * 16, 16)] = jnp.ones((16,), jnp.float32)
    for j in range(RPT // 16):
        stage_v[pl.ds(j * 16, 16)] = jnp.zeros((16,), jnp.float32)
    pltpu.sync_copy(stage_v, deg_acc.at[pl.ds(row0, RPT)])
    plsc.subcore_barrier()

    pltpu.async_copy(edges_hbm.at[pl.ds(ch0, K)], ibuf.at[0], isem)

    def group(t, ib):
        nib = 1 - ib

        @pl.when(t < SR_D - 1)
        def _():
            pltpu.async_copy(
                edges_hbm.at[pl.ds(ch0 + (t + 1) * K, K)], ibuf.at[nib], isem)

        pltpu.make_async_copy(
            edges_hbm.at[pl.ds(ch0, K)], ibuf.at[ib], isem).wait()
        for j in range(K):
            pltpu.async_copy(ones_v, deg_acc.at[ibuf.at[ib, j]], ssem, add=True)
        for j in range(K):
            pltpu.make_async_copy(
                ones_v, deg_acc.at[ibuf.at[ib, j]], ssem).wait()

    def body(u, carry):
        group(2 * u, 0)
        group(2 * u + 1, 1)
        return carry

    lax.fori_loop(0, SR_D // 2, body, 0)
    plsc.subcore_barrier()
    pltpu.sync_copy(deg_acc.at[pl.ds(row0, RPT)], stage_v)
    pltpu.sync_copy(stage_v, deg_out.at[c, pl.ds(row0, RPT)])


def _agg_body(h_hbm, src_hbm, dst_hbm, out_hbm,
              sbuf, dbuf, rows, acc, isem, gsem, ssem):
    c = lax.axis_index("c")
    s = lax.axis_index("s")
    wid = s * NC + c
    row0 = s * RPT
    ch0 = wid * CPW

    for i in range(16):
        for j in range(D // 16):
            rows[0, i, pl.ds(j * 16, 16)] = jnp.zeros((16,), jnp.float32)

    def zbody(k, carry):
        pltpu.sync_copy(rows.at[0, pl.ds(0, 16)],
                        acc.at[pl.ds(row0 + k * 16, 16)])
        return carry

    lax.fori_loop(0, RPT // 16, zbody, 0)
    plsc.subcore_barrier()

    pltpu.async_copy(src_hbm.at[pl.ds(ch0, K)], sbuf.at[0], isem)
    pltpu.async_copy(dst_hbm.at[pl.ds(ch0, K)], dbuf.at[0], isem)

    def group(t, ib):
        nib = 1 - ib

        @pl.when(t < SR_E - 1)
        def _():
            base = ch0 + (t + 1) * K
            pltpu.async_copy(src_hbm.at[pl.ds(base, K)], sbuf.at[nib], isem)
            pltpu.async_copy(dst_hbm.at[pl.ds(base, K)], dbuf.at[nib], isem)

        pltpu.make_async_copy(
            src_hbm.at[pl.ds(ch0, K)], sbuf.at[ib], isem).wait()
        pltpu.make_async_copy(
            dst_hbm.at[pl.ds(ch0, K)], dbuf.at[ib], isem).wait()

        pltpu.async_copy(h_hbm.at[sbuf.at[ib, 0]], rows.at[0], gsem)
        for j in range(K):
            b = j % 2
            if j + 1 < K:
                if j >= 1:
                    pltpu.make_async_copy(
                        rows.at[(j + 1) % 2],
                        acc.at[dbuf.at[ib, j - 1]], ssem).wait()
                pltpu.async_copy(
                    h_hbm.at[sbuf.at[ib, j + 1]], rows.at[(j + 1) % 2], gsem)
            pltpu.make_async_copy(
                h_hbm.at[sbuf.at[ib, j]], rows.at[b], gsem).wait()
            pltpu.async_copy(rows.at[b], acc.at[dbuf.at[ib, j]], ssem, add=True)
        pltpu.make_async_copy(
            rows.at[0], acc.at[dbuf.at[ib, K - 2]], ssem).wait()
        pltpu.make_async_copy(
            rows.at[1], acc.at[dbuf.at[ib, K - 1]], ssem).wait()

    def body(u, carry):
        group(2 * u, 0)
        group(2 * u + 1, 1)
        return carry

    lax.fori_loop(0, SR_E // 2, body, 0)
    plsc.subcore_barrier()
    for k in range(RPT // C):
        pltpu.sync_copy(acc.at[pl.ds(row0 + k * C, C)], rows.at[0])
        pltpu.sync_copy(rows.at[0], out_hbm.at[c, pl.ds(row0 + k * C, C)])


def _sc_degree(edges2d):
    mesh = plsc.VectorSubcoreMesh(core_axis_name="c", subcore_axis_name="s")
    f = pl.kernel(
        _deg_body,
        out_type=jax.ShapeDtypeStruct((NC, NPAD), jnp.float32),
        mesh=mesh,
        scratch_types=[
            pltpu.VMEM((2, K, C), jnp.int32),
            pltpu.VMEM((C,), jnp.float32),
            pltpu.VMEM((RPT,), jnp.float32),
            pltpu.VMEM_SHARED((NPAD,), jnp.float32),
            pltpu.SemaphoreType.DMA,
            pltpu.SemaphoreType.DMA,
        ],
    )
    return f(edges2d)


def _sc_aggregate(h, src2d, dst2d):
    mesh = plsc.VectorSubcoreMesh(core_axis_name="c", subcore_axis_name="s")
    f = pl.kernel(
        _agg_body,
        out_type=jax.ShapeDtypeStruct((NC, NPAD, D), jnp.float32),
        mesh=mesh,
        scratch_types=[
            pltpu.VMEM((2, K, C), jnp.int32),
            pltpu.VMEM((2, K, C), jnp.int32),
            pltpu.VMEM((2, C, D), jnp.float32),
            pltpu.VMEM_SHARED((NPAD, D), jnp.float32),
            pltpu.SemaphoreType.DMA,
            pltpu.SemaphoreType.DMA,
            pltpu.SemaphoreType.DMA,
        ],
    )
    return f(h, src2d, dst2d)


def _prep_body(deg_ref, emb_ref, norm_ref, h_ref):
    d = deg_ref[0] + deg_ref[1]
    n = lax.rsqrt(jnp.maximum(d, 1.0))
    norm_ref[...] = n
    h_ref[...] = emb_ref[...] * n


def _tc_prep(deg2, emb):
    return pl.pallas_call(
        _prep_body,
        grid=(TCG,),
        in_specs=[
            pl.BlockSpec((NC, TCB, 1), lambda i: (0, i, 0)),
            pl.BlockSpec((TCB, D), lambda i: (i, 0)),
        ],
        out_specs=[
            pl.BlockSpec((TCB, 1), lambda i: (i, 0)),
            pl.BlockSpec((TCB, D), lambda i: (i, 0)),
        ],
        out_shape=[
            jax.ShapeDtypeStruct((NPAD, 1), jnp.float32),
            jax.ShapeDtypeStruct((NPAD, D), jnp.float32),
        ],
    )(deg2, emb)


def _comb_body(scale, p_ref, norm_ref, s_ref, sout_ref, hout_ref):
    nrm = norm_ref[...]
    e = (p_ref[0] + p_ref[1]) * nrm
    sout_ref[...] = (s_ref[...] + e) * scale
    hout_ref[...] = e * nrm


def _tc_combine(p, norm, s_in, scale):
    return pl.pallas_call(
        functools.partial(_comb_body, scale),
        grid=(TCG,),
        in_specs=[
            pl.BlockSpec((NC, TCB, D), lambda i: (0, i, 0)),
            pl.BlockSpec((TCB, 1), lambda i: (i, 0)),
            pl.BlockSpec((TCB, D), lambda i: (i, 0)),
        ],
        out_specs=[
            pl.BlockSpec((TCB, D), lambda i: (i, 0)),
            pl.BlockSpec((TCB, D), lambda i: (i, 0)),
        ],
        out_shape=[
            jax.ShapeDtypeStruct((NPAD, D), jnp.float32),
            jax.ShapeDtypeStruct((NPAD, D), jnp.float32),
        ],
    )(p, norm, s_in)


def kernel(entity_embedding, edge_index):
    src = edge_index[0]
    dst = edge_index[1]
    pad_e = 10000 + (jnp.arange(NCH_E * C - E, dtype=jnp.int32) % (NPAD - N))
    pad_d = 10000 + (jnp.arange(NCH_D * C - 2 * E, dtype=jnp.int32) % (NPAD - N))
    src2d = jnp.concatenate([src, pad_e]).reshape(NCH_E, C)
    dst2d = jnp.concatenate([dst, pad_e]).reshape(NCH_E, C)
    edges2d = jnp.concatenate([edge_index.reshape(2 * E), pad_d]).reshape(NCH_D, C)
    emb = jnp.pad(entity_embedding, ((0, NPAD - N), (0, 0)))

    deg2 = _sc_degree(edges2d).reshape(NC, NPAD, 1)
    norm, h = _tc_prep(deg2, emb)

    s_acc = emb
    for layer in range(3):
        p = _sc_aggregate(h, src2d, dst2d)
        scale = 0.25 if layer == 2 else 1.0
        s_acc, h = _tc_combine(p, norm, s_acc, scale)

    return s_acc[:N]


# speedup vs baseline: 3.7831x; 1.0873x over previous
import functools

import jax
import jax.numpy as jnp
from jax import lax
from jax.experimental import pallas as pl
from jax.experimental.pallas import tpu as pltpu
from jax.experimental.pallas import tpu_sc as plsc

N = 10000
D = 128
E = 320000
NC, NS = 2, 16
NW = NC * NS
NPAD = 10240
RPT = NPAD // NS
C = 128
CA = 32
K = 32
NB = 8
KD = 8

CPW = 320
NCH_E = NW * CPW
SR_E = CPW // K

CPW_D = 160
NCH_D = NW * CPW_D
SR_D = CPW_D // KD

TCB = 2000
TCG = N // TCB


def _deg_body(edges_hbm, deg_out, ibuf, ones_v, stage_v, deg_acc, isem, ssem):
    c = lax.axis_index("c")
    s = lax.axis_index("s")
    wid = s * NC + c
    row0 = s * RPT
    ch0 = wid * CPW_D
    for j in range(C // 16):
        ones_v[pl.ds(j * 16, 16)] = jnp.ones((16,), jnp.float32)
    for j in range(RPT // 16):
        stage_v[pl.ds(j * 16, 16)] = jnp.zeros((16,), jnp.float32)
    pltpu.sync_copy(stage_v, deg_acc.at[pl.ds(row0, RPT)])
    plsc.subcore_barrier()

    pltpu.async_copy(edges_hbm.at[pl.ds(ch0, KD)], ibuf.at[0], isem)

    def group(t, ib):
        nib = 1 - ib

        @pl.when(t < SR_D - 1)
        def _():
            pltpu.async_copy(
                edges_hbm.at[pl.ds(ch0 + (t + 1) * KD, KD)], ibuf.at[nib], isem)

        pltpu.make_async_copy(
            edges_hbm.at[pl.ds(ch0, KD)], ibuf.at[ib], isem).wait()
        for j in range(KD):
            pltpu.async_copy(ones_v, deg_acc.at[ibuf.at[ib, j]], ssem, add=True)
        for j in range(KD):
            pltpu.make_async_copy(
                ones_v, deg_acc.at[ibuf.at[ib, j]], ssem).wait()

    def body(u, carry):
        group(2 * u, 0)
        group(2 * u + 1, 1)
        return carry

    lax.fori_loop(0, SR_D // 2, body, 0)
    plsc.subcore_barrier()
    pltpu.sync_copy(deg_acc.at[pl.ds(row0, RPT)], stage_v)
    pltpu.sync_copy(stage_v, deg_out.at[c, pl.ds(row0, RPT)])


def _agg_body(h_hbm, src_hbm, dst_hbm, out_hbm,
              sbuf, dbuf, rows, acc, isem, gsem, ssem):
    c = lax.axis_index("c")
    s = lax.axis_index("s")
    wid = s * NC + c
    row0 = s * RPT
    ch0 = wid * CPW

    def zrow(i, carry):
        for j in range(D // 16):
            rows[0, i, pl.ds(j * 16, 16)] = jnp.zeros((16,), jnp.float32)
        return carry

    lax.fori_loop(0, CA, zrow, 0)
    for k in range(RPT // CA):
        pltpu.async_copy(rows.at[0], acc.at[pl.ds(row0 + k * CA, CA)], gsem)
    for k in range(RPT // CA):
        pltpu.make_async_copy(
            rows.at[0], acc.at[pl.ds(row0 + k * CA, CA)], gsem).wait()
    plsc.subcore_barrier()

    pltpu.async_copy(src_hbm.at[pl.ds(ch0, K)], sbuf.at[0], isem)
    pltpu.async_copy(dst_hbm.at[pl.ds(ch0, K)], dbuf.at[0], isem)

    def group(t, ib):
        t = jnp.int32(t)
        nib = 1 - ib

        @pl.when(t < SR_E - 1)
        def _():
            base = ch0 + (t + 1) * K
            pltpu.async_copy(src_hbm.at[pl.ds(base, K)], sbuf.at[nib], isem)
            pltpu.async_copy(dst_hbm.at[pl.ds(base, K)], dbuf.at[nib], isem)

        pltpu.make_async_copy(
            src_hbm.at[pl.ds(ch0, K)], sbuf.at[ib], isem).wait()
        pltpu.make_async_copy(
            dst_hbm.at[pl.ds(ch0, K)], dbuf.at[ib], isem).wait()

        def issue_gather(m):
            pltpu.async_copy(h_hbm.at[sbuf.at[ib, m]], rows.at[m % NB], gsem)

        def wait_gather(m):
            pltpu.make_async_copy(
                h_hbm.at[sbuf.at[ib, m]], rows.at[m % NB], gsem).wait()

        def issue_scatter(m):
            pltpu.async_copy(
                rows.at[m % NB], acc.at[dbuf.at[ib, m]], ssem, add=True)

        def wait_scatter(m):
            pltpu.make_async_copy(
                rows.at[m % NB], acc.at[dbuf.at[ib, m]], ssem).wait()

        for m in range(NB - 1):
            issue_gather(m)
        for j in range(K):
            m = j + NB - 1
            if m < K:
                if m >= NB:
                    wait_scatter(m - NB)
                issue_gather(m)
            wait_gather(j)
            issue_scatter(j)
        for d in range(K - NB, K):
            wait_scatter(d)

    def body(u, carry):
        group(2 * u, 0)
        group(2 * u + 1, 1)
        return carry

    lax.fori_loop(0, SR_E // 2, body, 0)
    if SR_E % 2:
        group(SR_E - 1, (SR_E - 1) % 2)
    plsc.subcore_barrier()

    NO = RPT // CA

    def rd(k):
        return (acc.at[pl.ds(row0 + k * CA, CA)], rows.at[k % NB])

    def wr(k):
        return (rows.at[k % NB], out_hbm.at[c, pl.ds(row0 + k * CA, CA)])

    for m in range(NB - 1):
        pltpu.async_copy(*rd(m), gsem)
    for k in range(NO):
        m = k + NB - 1
        if m < NO:
            if m >= NB:
                pltpu.make_async_copy(*wr(m - NB), ssem).wait()
            pltpu.async_copy(*rd(m), gsem)
        pltpu.make_async_copy(*rd(k), gsem).wait()
        pltpu.async_copy(*wr(k), ssem, add=False)
    for d in range(NO - NB, NO):
        pltpu.make_async_copy(*wr(d), ssem).wait()


def _sc_degree(edges2d):
    mesh = plsc.VectorSubcoreMesh(core_axis_name="c", subcore_axis_name="s")
    f = pl.kernel(
        _deg_body,
        out_type=jax.ShapeDtypeStruct((NC, NPAD), jnp.float32),
        mesh=mesh,
        scratch_types=[
            pltpu.VMEM((2, KD, C), jnp.int32),
            pltpu.VMEM((C,), jnp.float32),
            pltpu.VMEM((RPT,), jnp.float32),
            pltpu.VMEM_SHARED((NPAD,), jnp.float32),
            pltpu.SemaphoreType.DMA,
            pltpu.SemaphoreType.DMA,
        ],
    )
    return f(edges2d)


def _sc_aggregate(h, src2d, dst2d):
    mesh = plsc.VectorSubcoreMesh(core_axis_name="c", subcore_axis_name="s")
    f = pl.kernel(
        _agg_body,
        out_type=jax.ShapeDtypeStruct((NC, NPAD, D), jnp.float32),
        mesh=mesh,
        scratch_types=[
            pltpu.VMEM((2, K, CA), jnp.int32),
            pltpu.VMEM((2, K, CA), jnp.int32),
            pltpu.VMEM((NB, CA, D), jnp.float32),
            pltpu.VMEM_SHARED((NPAD, D), jnp.float32),
            pltpu.SemaphoreType.DMA,
            pltpu.SemaphoreType.DMA,
            pltpu.SemaphoreType.DMA,
        ],
    )
    return f(h, src2d, dst2d)


def _prep_body(deg_ref, emb_ref, norm_ref, h_ref):
    d = deg_ref[0] + deg_ref[1]
    n = lax.rsqrt(jnp.maximum(d, 1.0))
    norm_ref[...] = n
    h_ref[...] = emb_ref[...] * n


def _tc_prep(deg2, emb):
    return pl.pallas_call(
        _prep_body,
        grid=(TCG,),
        in_specs=[
            pl.BlockSpec((NC, TCB, 1), lambda i: (0, i, 0)),
            pl.BlockSpec((TCB, D), lambda i: (i, 0)),
        ],
        out_specs=[
            pl.BlockSpec((TCB, 1), lambda i: (i, 0)),
            pl.BlockSpec((TCB, D), lambda i: (i, 0)),
        ],
        out_shape=[
            jax.ShapeDtypeStruct((N, 1), jnp.float32),
            jax.ShapeDtypeStruct((N, D), jnp.float32),
        ],
    )(deg2, emb)


def _comb_body(scale, p_ref, norm_ref, s_ref, sout_ref, hout_ref):
    nrm = norm_ref[...]
    e = (p_ref[0] + p_ref[1]) * nrm
    sout_ref[...] = (s_ref[...] + e) * scale
    hout_ref[...] = e * nrm


def _tc_combine(p, norm, s_in, scale):
    return pl.pallas_call(
        functools.partial(_comb_body, scale),
        grid=(TCG,),
        in_specs=[
            pl.BlockSpec((NC, TCB, D), lambda i: (0, i, 0)),
            pl.BlockSpec((TCB, 1), lambda i: (i, 0)),
            pl.BlockSpec((TCB, D), lambda i: (i, 0)),
        ],
        out_specs=[
            pl.BlockSpec((TCB, D), lambda i: (i, 0)),
            pl.BlockSpec((TCB, D), lambda i: (i, 0)),
        ],
        out_shape=[
            jax.ShapeDtypeStruct((N, D), jnp.float32),
            jax.ShapeDtypeStruct((N, D), jnp.float32),
        ],
    )(p, norm, s_in)


def kernel(entity_embedding, edge_index):
    src = edge_index[0]
    dst = edge_index[1]
    pad_s = jnp.arange(NCH_E * CA - E, dtype=jnp.int32) % N
    pad_e = N + (jnp.arange(NCH_E * CA - E, dtype=jnp.int32) % (NPAD - N))
    pad_d = N + (jnp.arange(NCH_D * C - 2 * E, dtype=jnp.int32) % (NPAD - N))
    src2d = jnp.concatenate([src, pad_s]).reshape(NCH_E, CA)
    dst2d = jnp.concatenate([dst, pad_e]).reshape(NCH_E, CA)
    edges2d = jnp.concatenate([edge_index.reshape(2 * E), pad_d]).reshape(NCH_D, C)

    deg2 = _sc_degree(edges2d).reshape(NC, NPAD, 1)
    norm, h = _tc_prep(deg2, entity_embedding)

    s_acc = entity_embedding
    for layer in range(3):
        p = _sc_aggregate(h, src2d, dst2d)
        scale = 0.25 if layer == 2 else 1.0
        s_acc, h = _tc_combine(p, norm, s_acc, scale)

    return s_acc


# speedup vs baseline: 4.1071x; 1.0856x over previous
import functools

import jax
import jax.numpy as jnp
from jax import lax
from jax.experimental import pallas as pl
from jax.experimental.pallas import tpu as pltpu
from jax.experimental.pallas import tpu_sc as plsc

N = 10000
D = 128
E = 320000
NC, NS = 2, 16
NW = NC * NS
NPAD = 10240
RPT = NPAD // NS
C = 128
CA = 64
K = 32
NB = 4
KD = 8

CPW = 160
NCH_E = NW * CPW
SR_E = CPW // K

CPW_D = 160
NCH_D = NW * CPW_D
SR_D = CPW_D // KD

TCB = 2000
TCG = N // TCB


def _deg_body(edges_hbm, deg_out, ibuf, ones_v, stage_v, deg_acc, isem, ssem):
    c = lax.axis_index("c")
    s = lax.axis_index("s")
    wid = s * NC + c
    row0 = s * RPT
    ch0 = wid * CPW_D
    for j in range(C // 16):
        ones_v[pl.ds(j * 16, 16)] = jnp.ones((16,), jnp.float32)
    for j in range(RPT // 16):
        stage_v[pl.ds(j * 16, 16)] = jnp.zeros((16,), jnp.float32)
    pltpu.sync_copy(stage_v, deg_acc.at[pl.ds(row0, RPT)])
    plsc.subcore_barrier()

    pltpu.async_copy(edges_hbm.at[pl.ds(ch0, KD)], ibuf.at[0], isem)

    def group(t, ib):
        nib = 1 - ib

        @pl.when(t < SR_D - 1)
        def _():
            pltpu.async_copy(
                edges_hbm.at[pl.ds(ch0 + (t + 1) * KD, KD)], ibuf.at[nib], isem)

        pltpu.make_async_copy(
            edges_hbm.at[pl.ds(ch0, KD)], ibuf.at[ib], isem).wait()
        for j in range(KD):
            pltpu.async_copy(ones_v, deg_acc.at[ibuf.at[ib, j]], ssem, add=True)
        for j in range(KD):
            pltpu.make_async_copy(
                ones_v, deg_acc.at[ibuf.at[ib, j]], ssem).wait()

    def body(u, carry):
        group(2 * u, 0)
        group(2 * u + 1, 1)
        return carry

    lax.fori_loop(0, SR_D // 2, body, 0)
    plsc.subcore_barrier()
    pltpu.sync_copy(deg_acc.at[pl.ds(row0, RPT)], stage_v)
    pltpu.sync_copy(stage_v, deg_out.at[c, pl.ds(row0, RPT)])


def _agg_body(h_hbm, src_hbm, dst_hbm, out_hbm,
              sbuf, dbuf, rows, acc, isem, gsem, ssem):
    c = lax.axis_index("c")
    s = lax.axis_index("s")
    wid = s * NC + c
    row0 = s * RPT
    ch0 = wid * CPW

    def zrow(i, carry):
        for j in range(D // 16):
            rows[0, i, pl.ds(j * 16, 16)] = jnp.zeros((16,), jnp.float32)
        return carry

    lax.fori_loop(0, CA, zrow, 0)
    for k in range(RPT // CA):
        pltpu.async_copy(rows.at[0], acc.at[pl.ds(row0 + k * CA, CA)], gsem)
    for k in range(RPT // CA):
        pltpu.make_async_copy(
            rows.at[0], acc.at[pl.ds(row0 + k * CA, CA)], gsem).wait()
    plsc.subcore_barrier()

    pltpu.async_copy(src_hbm.at[pl.ds(ch0, K)], sbuf.at[0], isem)
    pltpu.async_copy(dst_hbm.at[pl.ds(ch0, K)], dbuf.at[0], isem)

    def group(t, ib):
        t = jnp.int32(t)
        nib = 1 - ib

        @pl.when(t < SR_E - 1)
        def _():
            base = ch0 + (t + 1) * K
            pltpu.async_copy(src_hbm.at[pl.ds(base, K)], sbuf.at[nib], isem)
            pltpu.async_copy(dst_hbm.at[pl.ds(base, K)], dbuf.at[nib], isem)

        pltpu.make_async_copy(
            src_hbm.at[pl.ds(ch0, K)], sbuf.at[ib], isem).wait()
        pltpu.make_async_copy(
            dst_hbm.at[pl.ds(ch0, K)], dbuf.at[ib], isem).wait()

        def issue_gather(m):
            pltpu.async_copy(h_hbm.at[sbuf.at[ib, m]], rows.at[m % NB], gsem)

        def wait_gather(m):
            pltpu.make_async_copy(
                h_hbm.at[sbuf.at[ib, m]], rows.at[m % NB], gsem).wait()

        def issue_scatter(m):
            pltpu.async_copy(
                rows.at[m % NB], acc.at[dbuf.at[ib, m]], ssem, add=True)

        def wait_scatter(m):
            pltpu.make_async_copy(
                rows.at[m % NB], acc.at[dbuf.at[ib, m]], ssem).wait()

        for m in range(NB - 1):
            issue_gather(m)
        for j in range(K):
            m = j + NB - 1
            if m < K:
                if m >= NB:
                    wait_scatter(m - NB)
                issue_gather(m)
            wait_gather(j)
            issue_scatter(j)
        for d in range(K - NB, K):
            wait_scatter(d)

    def body(u, carry):
        group(2 * u, 0)
        group(2 * u + 1, 1)
        return carry

    lax.fori_loop(0, SR_E // 2, body, 0)
    if SR_E % 2:
        group(SR_E - 1, (SR_E - 1) % 2)
    plsc.subcore_barrier()

    NO = RPT // CA

    def rd(k):
        return (acc.at[pl.ds(row0 + k * CA, CA)], rows.at[k % NB])

    def wr(k):
        return (rows.at[k % NB], out_hbm.at[c, pl.ds(row0 + k * CA, CA)])

    for m in range(NB - 1):
        pltpu.async_copy(*rd(m), gsem)
    for k in range(NO):
        m = k + NB - 1
        if m < NO:
            if m >= NB:
                pltpu.make_async_copy(*wr(m - NB), ssem).wait()
            pltpu.async_copy(*rd(m), gsem)
        pltpu.make_async_copy(*rd(k), gsem).wait()
        pltpu.async_copy(*wr(k), ssem, add=False)
    for d in range(NO - NB, NO):
        pltpu.make_async_copy(*wr(d), ssem).wait()


def _sc_degree(edges2d):
    mesh = plsc.VectorSubcoreMesh(core_axis_name="c", subcore_axis_name="s")
    f = pl.kernel(
        _deg_body,
        out_type=jax.ShapeDtypeStruct((NC, NPAD), jnp.float32),
        mesh=mesh,
        scratch_types=[
            pltpu.VMEM((2, KD, C), jnp.int32),
            pltpu.VMEM((C,), jnp.float32),
            pltpu.VMEM((RPT,), jnp.float32),
            pltpu.VMEM_SHARED((NPAD,), jnp.float32),
            pltpu.SemaphoreType.DMA,
            pltpu.SemaphoreType.DMA,
        ],
    )
    return f(edges2d)


def _sc_aggregate(h, src2d, dst2d):
    mesh = plsc.VectorSubcoreMesh(core_axis_name="c", subcore_axis_name="s")
    f = pl.kernel(
        _agg_body,
        out_type=jax.ShapeDtypeStruct((NC, NPAD, D), jnp.float32),
        mesh=mesh,
        scratch_types=[
            pltpu.VMEM((2, K, CA), jnp.int32),
            pltpu.VMEM((2, K, CA), jnp.int32),
            pltpu.VMEM((NB, CA, D), jnp.float32),
            pltpu.VMEM_SHARED((NPAD, D), jnp.float32),
            pltpu.SemaphoreType.DMA,
            pltpu.SemaphoreType.DMA,
            pltpu.SemaphoreType.DMA,
        ],
    )
    return f(h, src2d, dst2d)


def _prep_body(deg_ref, emb_ref, norm_ref, h_ref):
    d = deg_ref[0] + deg_ref[1]
    n = lax.rsqrt(jnp.maximum(d, 1.0))
    norm_ref[...] = n
    h_ref[...] = emb_ref[...] * n


def _tc_prep(deg2, emb):
    return pl.pallas_call(
        _prep_body,
        grid=(TCG,),
        in_specs=[
            pl.BlockSpec((NC, TCB, 1), lambda i: (0, i, 0)),
            pl.BlockSpec((TCB, D), lambda i: (i, 0)),
        ],
        out_specs=[
            pl.BlockSpec((TCB, 1), lambda i: (i, 0)),
            pl.BlockSpec((TCB, D), lambda i: (i, 0)),
        ],
        out_shape=[
            jax.ShapeDtypeStruct((N, 1), jnp.float32),
            jax.ShapeDtypeStruct((N, D), jnp.float32),
        ],
    )(deg2, emb)


def _comb_body(scale, p_ref, norm_ref, s_ref, sout_ref, hout_ref):
    nrm = norm_ref[...]
    e = (p_ref[0] + p_ref[1]) * nrm
    sout_ref[...] = (s_ref[...] + e) * scale
    hout_ref[...] = e * nrm


def _final_body(p_ref, norm_ref, s_ref, sout_ref):
    e = (p_ref[0] + p_ref[1]) * norm_ref[...]
    sout_ref[...] = (s_ref[...] + e) * 0.25


def _tc_final(p, norm, s_in):
    return pl.pallas_call(
        _final_body,
        grid=(TCG,),
        in_specs=[
            pl.BlockSpec((NC, TCB, D), lambda i: (0, i, 0)),
            pl.BlockSpec((TCB, 1), lambda i: (i, 0)),
            pl.BlockSpec((TCB, D), lambda i: (i, 0)),
        ],
        out_specs=pl.BlockSpec((TCB, D), lambda i: (i, 0)),
        out_shape=jax.ShapeDtypeStruct((N, D), jnp.float32),
    )(p, norm, s_in)


def _tc_combine(p, norm, s_in, scale):
    return pl.pallas_call(
        functools.partial(_comb_body, scale),
        grid=(TCG,),
        in_specs=[
            pl.BlockSpec((NC, TCB, D), lambda i: (0, i, 0)),
            pl.BlockSpec((TCB, 1), lambda i: (i, 0)),
            pl.BlockSpec((TCB, D), lambda i: (i, 0)),
        ],
        out_specs=[
            pl.BlockSpec((TCB, D), lambda i: (i, 0)),
            pl.BlockSpec((TCB, D), lambda i: (i, 0)),
        ],
        out_shape=[
            jax.ShapeDtypeStruct((N, D), jnp.float32),
            jax.ShapeDtypeStruct((N, D), jnp.float32),
        ],
    )(p, norm, s_in)


def kernel(entity_embedding, edge_index):
    src = edge_index[0]
    dst = edge_index[1]
    pad_s = jnp.arange(NCH_E * CA - E, dtype=jnp.int32) % N
    pad_e = N + (jnp.arange(NCH_E * CA - E, dtype=jnp.int32) % (NPAD - N))
    pad_d = N + (jnp.arange(NCH_D * C - 2 * E, dtype=jnp.int32) % (NPAD - N))
    src2d = jnp.concatenate([src, pad_s]).reshape(NCH_E, CA)
    dst2d = jnp.concatenate([dst, pad_e]).reshape(NCH_E, CA)
    edges2d = jnp.concatenate([edge_index.reshape(2 * E), pad_d]).reshape(NCH_D, C)

    deg2 = _sc_degree(edges2d).reshape(NC, NPAD, 1)
    norm, h = _tc_prep(deg2, entity_embedding)

    s_acc = entity_embedding
    for layer in range(2):
        p = _sc_aggregate(h, src2d, dst2d)
        s_acc, h = _tc_combine(p, norm, s_acc, 1.0)
    p = _sc_aggregate(h, src2d, dst2d)
    return _tc_final(p, norm, s_acc)
